# baseline (speedup 1.0000x reference)
import functools

import jax
import jax.numpy as jnp
from jax import lax
from jax.experimental import pallas as pl
from jax.experimental.pallas import tpu as pltpu
from jax.experimental.pallas import tpu_sc as plsc

_LANES = 16
_WINDOW = 128
_NW = 32
_CPIX = 1024


def _sc_gather_rows(table, idx):
    n, d = idx.shape[0], table.shape[1]
    nwin = n // _WINDOW
    mesh = plsc.VectorSubcoreMesh(core_axis_name="c", subcore_axis_name="s")

    @functools.partial(
        pl.kernel,
        out_type=jax.ShapeDtypeStruct((n, d), table.dtype),
        mesh=mesh,
        compiler_params=pltpu.CompilerParams(use_tc_tiling_on_sc=False),
    )
    def gather_kernel(table_hbm, idx_hbm, out_hbm):
        def body(idx_vmem, out_vmem):
            pltpu.sync_copy(table_hbm.at[idx_vmem.at[0]], out_vmem)

        pltpu.emit_pipeline(
            body,
            grid=(nwin,),
            in_specs=[pl.BlockSpec((1, _WINDOW), lambda i: (0, i))],
            out_specs=[pl.BlockSpec((_WINDOW, d), lambda i: (i, 0))],
            core_axis_name=("c", "s"),
            dimension_semantics=(pltpu.PARALLEL,),
        )(idx_hbm, out_hbm)

    return gather_kernel(table, idx.reshape(1, n))


def _sc_dist2(tbl, idx, bary_flat, npix):
    mesh = plsc.VectorSubcoreMesh(core_axis_name="c", subcore_axis_name="s")
    chunks = npix // _CPIX
    per_w = chunks // _NW
    ch = _CPIX * 3
    ngath = ch // _WINDOW
    out_t = jax.ShapeDtypeStruct((npix,), jnp.float32)

    @functools.partial(
        pl.kernel,
        out_type=(out_t, out_t, out_t),
        mesh=mesh,
        compiler_params=pltpu.CompilerParams(
            use_tc_tiling_on_sc=False, needs_layout_passes=False),
        scratch_types=[
            pltpu.VMEM((ch,), jnp.int32),
            pltpu.VMEM((ch, _LANES), jnp.float32),
            pltpu.VMEM((_CPIX * 9,), jnp.float32),
            pltpu.VMEM((_CPIX,), jnp.float32),
            pltpu.VMEM((_CPIX,), jnp.float32),
            pltpu.VMEM((_CPIX,), jnp.float32),
            pltpu.SemaphoreType.DMA,
        ],
    )
    def dist2_kernel(tbl_hbm, idx_hbm, bary_hbm, o0_hbm, o1_hbm, o2_hbm,
                     idx_v, rows_v, bary_v, d0_v, d1_v, d2_v, gsem):
        wid = lax.axis_index("s") * 2 + lax.axis_index("c")
        outs = (o0_hbm, o1_hbm, o2_hbm)
        dvs = (d0_v, d1_v, d2_v)

        @pl.loop(0, per_w)
        def _(t):
            chunk = wid * per_w + t
            pbase = chunk * _CPIX
            hbase = pbase * 3
            pltpu.sync_copy(idx_hbm.at[pl.ds(hbase, ch)], idx_v)
            pltpu.sync_copy(bary_hbm.at[pl.ds(pbase * 9, _CPIX * 9)], bary_v)
            copies = [
                pltpu.async_copy(
                    tbl_hbm.at[idx_v.at[pl.ds(r * _WINDOW, _WINDOW)]],
                    rows_v.at[pl.ds(r * _WINDOW, _WINDOW)],
                    gsem)
                for r in range(ngath)
            ]
            for c in copies:
                c.wait()

            ci = lax.iota(jnp.int32, 16)

            @pl.loop(0, _CPIX // 16)
            def _(i):
                pix = i * 16 + ci
                for k in range(3):
                    row = pix * 3 + k
                    xs = [plsc.load_gather(
                              rows_v, [row, jnp.full((16,), tt, jnp.int32)])
                          for tt in range(9)]
                    bs = [plsc.load_gather(bary_v, [pix * 9 + (3 * k + j)])
                          for j in range(3)]
                    p0 = bs[0] * xs[0] + bs[1] * xs[3] + bs[2] * xs[6]
                    p1 = bs[0] * xs[1] + bs[1] * xs[4] + bs[2] * xs[7]
                    p2 = bs[0] * xs[2] + bs[1] * xs[5] + bs[2] * xs[8]
                    dvs[k][pl.ds(i * 16, 16)] = p0 * p0 + p1 * p1 + p2 * p2

            for k in range(3):
                pltpu.sync_copy(dvs[k], outs[k].at[pl.ds(pbase, _CPIX)])

    return dist2_kernel(tbl, idx, bary_flat)


def _sqrt_body(x_ref, o_ref):
    o_ref[...] = jnp.sqrt(x_ref[...])


def _sqrt3(d2s, npix):
    blk = 65536
    outs = []
    for d2 in d2s:
        outs.append(pl.pallas_call(
            _sqrt_body,
            grid=(npix // blk,),
            in_specs=[pl.BlockSpec((blk,), lambda i: (i,))],
            out_specs=pl.BlockSpec((blk,), lambda i: (i,)),
            out_shape=jax.ShapeDtypeStruct((npix,), jnp.float32),
        )(d2))
    return outs


def kernel(pix_to_face, bary_coords, verts, faces):
    n, h, w, k = pix_to_face.shape
    f = faces.shape[0]
    npix = n * h * w

    verts_pad = jnp.pad(verts.astype(jnp.float32), ((0, 0), (0, _LANES - 3)))
    faces32 = faces.astype(jnp.int32)
    fp = ((f + _WINDOW - 1) // _WINDOW) * _WINDOW
    faces_pad = jnp.pad(faces32, ((0, fp - f), (0, 0)))
    corner_idx = faces_pad.T.reshape(-1)
    corner_rows = _sc_gather_rows(verts_pad, corner_idx)
    tbl = jnp.concatenate(
        [corner_rows[0 * fp:0 * fp + f, 0:3],
         corner_rows[1 * fp:1 * fp + f, 0:3],
         corner_rows[2 * fp:2 * fp + f, 0:3],
         jnp.zeros((f, _LANES - 9), jnp.float32)], axis=1)

    idx = pix_to_face.astype(jnp.int32).reshape(-1)
    bary_flat = bary_coords.astype(jnp.float32).reshape(-1)
    d2s = _sc_dist2(tbl, idx, bary_flat, npix)

    ds = _sqrt3(d2s, npix)
    return tuple(d.reshape(n, h, w, 1) for d in ds)


# speedup vs baseline: 1.9021x; 1.9021x over previous
import functools

import jax
import jax.numpy as jnp
from jax.experimental import pallas as pl
from jax.experimental.pallas import tpu as pltpu
from jax.experimental.pallas import tpu_sc as plsc

_LANES = 32
_WINDOW = 128


def _sc_gather_rows(table, idx):
    n, d = idx.shape[0], table.shape[1]
    nwin = n // _WINDOW
    mesh = plsc.VectorSubcoreMesh(core_axis_name="c", subcore_axis_name="s")

    @functools.partial(
        pl.kernel,
        out_type=jax.ShapeDtypeStruct((n, d), table.dtype),
        mesh=mesh,
        compiler_params=pltpu.CompilerParams(use_tc_tiling_on_sc=False),
    )
    def gather_kernel(table_hbm, idx_hbm, out_hbm):
        def body(idx_vmem, out_vmem):
            pltpu.sync_copy(table_hbm.at[idx_vmem.at[0]], out_vmem)

        pltpu.emit_pipeline(
            body,
            grid=(nwin,),
            in_specs=[pl.BlockSpec((1, _WINDOW), lambda i: (0, i))],
            out_specs=[pl.BlockSpec((_WINDOW, d), lambda i: (i, 0))],
            core_axis_name=("c", "s"),
            dimension_semantics=(pltpu.PARALLEL,),
        )(idx_hbm, out_hbm)

    return gather_kernel(table, idx.reshape(1, n))


def _dist_body(gt_ref, bt_ref, o_ref):
    b = [bt_ref[j].astype(jnp.float32) for j in range(3)]
    x = [gt_ref[t].astype(jnp.float32) for t in range(9)]
    p0 = b[0] * x[0] + b[1] * x[3] + b[2] * x[6]
    p1 = b[0] * x[1] + b[1] * x[4] + b[2] * x[7]
    p2 = b[0] * x[2] + b[1] * x[5] + b[2] * x[8]
    o_ref[...] = jnp.sqrt(p0 * p0 + p1 * p1 + p2 * p2)


def _dist(gt, bt, total):
    cols = 1024
    rows = total // cols
    br = 128
    out = pl.pallas_call(
        _dist_body,
        grid=(rows // br,),
        in_specs=[
            pl.BlockSpec((9, br, cols), lambda i: (0, i, 0)),
            pl.BlockSpec((3, br, cols), lambda i: (0, i, 0)),
        ],
        out_specs=pl.BlockSpec((br, cols), lambda i: (i, 0)),
        out_shape=jax.ShapeDtypeStruct((rows, cols), jnp.float32),
    )(gt.reshape(9, rows, cols), bt.reshape(3, rows, cols))
    return out.reshape(total)


def kernel(pix_to_face, bary_coords, verts, faces):
    n, h, w, k = pix_to_face.shape
    f = faces.shape[0]
    b = n * h * w * k

    verts_pad = jnp.pad(verts.astype(jnp.bfloat16), ((0, 0), (0, _LANES - 3)))
    faces32 = faces.astype(jnp.int32)
    fp = ((f + _WINDOW - 1) // _WINDOW) * _WINDOW
    faces_pad = jnp.pad(faces32, ((0, fp - f), (0, 0)))
    corner_idx = faces_pad.T.reshape(-1)
    corner_rows = _sc_gather_rows(verts_pad, corner_idx)
    tbl = jnp.concatenate(
        [corner_rows[0 * fp:0 * fp + f, 0:3],
         corner_rows[1 * fp:1 * fp + f, 0:3],
         corner_rows[2 * fp:2 * fp + f, 0:3],
         jnp.zeros((f, _LANES - 9), jnp.bfloat16)], axis=1)

    idx = pix_to_face.astype(jnp.int32).transpose(3, 0, 1, 2).reshape(-1)
    g = _sc_gather_rows(tbl, idx)

    gt = g.T[:9]
    bt = bary_coords.astype(jnp.bfloat16).transpose(4, 3, 0, 1, 2).reshape(3, b)
    d = _dist(gt, bt, b).reshape(k, n, h, w)
    return tuple(d[i].reshape(n, h, w, 1) for i in range(k))


# speedup vs baseline: 2.7393x; 1.4401x over previous
import functools

import jax
import jax.numpy as jnp
from jax.experimental import pallas as pl
from jax.experimental.pallas import tpu as pltpu
from jax.experimental.pallas import tpu_sc as plsc

_LANES = 32
_WINDOW = 128


def _sc_gather_rows(table, idx):
    n, d = idx.shape[0], table.shape[1]
    nwin = n // _WINDOW
    mesh = plsc.VectorSubcoreMesh(core_axis_name="c", subcore_axis_name="s")

    @functools.partial(
        pl.kernel,
        out_type=jax.ShapeDtypeStruct((n, d), table.dtype),
        mesh=mesh,
        compiler_params=pltpu.CompilerParams(use_tc_tiling_on_sc=False),
    )
    def gather_kernel(table_hbm, idx_hbm, out_hbm):
        def body(idx_vmem, out_vmem):
            pltpu.sync_copy(table_hbm.at[idx_vmem.at[0]], out_vmem)

        pltpu.emit_pipeline(
            body,
            grid=(nwin,),
            in_specs=[pl.BlockSpec((1, _WINDOW), lambda i: (0, i))],
            out_specs=[pl.BlockSpec((_WINDOW, d), lambda i: (i, 0))],
            core_axis_name=("c", "s"),
            dimension_semantics=(pltpu.PARALLEL,),
        )(idx_hbm, out_hbm)

    return gather_kernel(table, idx.reshape(1, n))


def _dist_body(gt_ref, bt_ref, o_ref):
    b = [bt_ref[j] for j in range(3)]
    x = [gt_ref[t].astype(jnp.float32) for t in range(9)]
    p0 = b[0] * x[0] + b[1] * x[3] + b[2] * x[6]
    p1 = b[0] * x[1] + b[1] * x[4] + b[2] * x[7]
    p2 = b[0] * x[2] + b[1] * x[5] + b[2] * x[8]
    o_ref[...] = jnp.sqrt(p0 * p0 + p1 * p1 + p2 * p2)


def _dist(gt, bt, total):
    cols = 1024
    rows = total // cols
    br = 128
    out = pl.pallas_call(
        _dist_body,
        grid=(rows // br,),
        in_specs=[
            pl.BlockSpec((9, br, cols), lambda i: (0, i, 0)),
            pl.BlockSpec((3, br, cols), lambda i: (0, i, 0)),
        ],
        out_specs=pl.BlockSpec((br, cols), lambda i: (i, 0)),
        out_shape=jax.ShapeDtypeStruct((rows, cols), jnp.float32),
    )(gt.reshape(9, rows, cols), bt.reshape(3, rows, cols))
    return out.reshape(total)


def kernel(pix_to_face, bary_coords, verts, faces):
    n, h, w, k = pix_to_face.shape
    f = faces.shape[0]
    b = n * h * w * k

    verts_pad = jnp.pad(verts.astype(jnp.bfloat16), ((0, 0), (0, _LANES - 3)))
    faces32 = faces.astype(jnp.int32)
    fp = ((f + _WINDOW - 1) // _WINDOW) * _WINDOW
    faces_pad = jnp.pad(faces32, ((0, fp - f), (0, 0)))
    corner_idx = faces_pad.T.reshape(-1)
    corner_rows = _sc_gather_rows(verts_pad, corner_idx)
    tbl = jnp.concatenate(
        [corner_rows[0 * fp:0 * fp + f, 0:3],
         corner_rows[1 * fp:1 * fp + f, 0:3],
         corner_rows[2 * fp:2 * fp + f, 0:3],
         jnp.zeros((f, _LANES - 9), jnp.bfloat16)], axis=1)

    idx = pix_to_face.astype(jnp.int32).transpose(3, 0, 1, 2).reshape(-1)
    g = _sc_gather_rows(tbl, idx)

    gt = g.T[:9]
    bt = bary_coords.astype(jnp.float32).transpose(4, 3, 0, 1, 2).reshape(3, b)
    d = _dist(gt, bt, b).reshape(k, n, h, w)
    return tuple(d[i].reshape(n, h, w, 1) for i in range(k))


# speedup vs baseline: 2.8429x; 1.0378x over previous
import functools

import jax
import jax.numpy as jnp
from jax.experimental import pallas as pl
from jax.experimental.pallas import tpu as pltpu
from jax.experimental.pallas import tpu_sc as plsc

_LANES = 16
_WINDOW = 128


def _sc_gather_rows(table, idx):
    n, d = idx.shape[0], table.shape[1]
    nwin = n // _WINDOW
    mesh = plsc.VectorSubcoreMesh(core_axis_name="c", subcore_axis_name="s")

    @functools.partial(
        pl.kernel,
        out_type=jax.ShapeDtypeStruct((n, d), table.dtype),
        mesh=mesh,
        compiler_params=pltpu.CompilerParams(use_tc_tiling_on_sc=False),
    )
    def gather_kernel(table_hbm, idx_hbm, out_hbm):
        def body(idx_vmem, out_vmem):
            pltpu.sync_copy(table_hbm.at[idx_vmem.at[0]], out_vmem)

        pltpu.emit_pipeline(
            body,
            grid=(nwin,),
            in_specs=[pl.BlockSpec((1, _WINDOW), lambda i: (0, i))],
            out_specs=[pl.BlockSpec((_WINDOW, d), lambda i: (i, 0))],
            core_axis_name=("c", "s"),
            dimension_semantics=(pltpu.PARALLEL,),
        )(idx_hbm, out_hbm)

    return gather_kernel(table, idx.reshape(1, n))


def _dist_body(gt_ref, bt_ref, o_ref):
    b0, b1, b2 = bt_ref[0], bt_ref[1], bt_ref[2]
    p0 = b0 * gt_ref[0] + b1 * gt_ref[3] + b2 * gt_ref[6]
    p1 = b0 * gt_ref[1] + b1 * gt_ref[4] + b2 * gt_ref[7]
    p2 = b0 * gt_ref[2] + b1 * gt_ref[5] + b2 * gt_ref[8]
    o_ref[...] = jnp.sqrt(p0 * p0 + p1 * p1 + p2 * p2)


def _dist(gt, bt, total):
    cols = 1024
    rows = total // cols
    br = 128
    out = pl.pallas_call(
        _dist_body,
        grid=(rows // br,),
        in_specs=[
            pl.BlockSpec((9, br, cols), lambda i: (0, i, 0)),
            pl.BlockSpec((3, br, cols), lambda i: (0, i, 0)),
        ],
        out_specs=pl.BlockSpec((br, cols), lambda i: (i, 0)),
        out_shape=jax.ShapeDtypeStruct((rows, cols), jnp.float32),
    )(gt.reshape(gt.shape[0], rows, cols), bt.reshape(3, rows, cols))
    return out.reshape(total)


def kernel(pix_to_face, bary_coords, verts, faces):
    n, h, w, k = pix_to_face.shape
    f = faces.shape[0]
    b = n * h * w * k

    verts_pad = jnp.pad(verts.astype(jnp.float32), ((0, 0), (0, _LANES - 3)))
    faces32 = faces.astype(jnp.int32)
    fp = ((f + _WINDOW - 1) // _WINDOW) * _WINDOW
    faces_pad = jnp.pad(faces32, ((0, fp - f), (0, 0)))
    corner_idx = faces_pad.T.reshape(-1)
    corner_rows = _sc_gather_rows(verts_pad, corner_idx)
    tbl = jnp.concatenate(
        [corner_rows[0 * fp:0 * fp + f, 0:3],
         corner_rows[1 * fp:1 * fp + f, 0:3],
         corner_rows[2 * fp:2 * fp + f, 0:3],
         jnp.zeros((f, _LANES - 9), jnp.float32)], axis=1)

    idx = pix_to_face.astype(jnp.int32).transpose(3, 0, 1, 2).reshape(-1)
    g = _sc_gather_rows(tbl, idx)

    gt = g.T[:9]
    bt = bary_coords.astype(jnp.float32).transpose(4, 3, 0, 1, 2).reshape(3, b)
    d = _dist(gt, bt, b).reshape(k, n, h, w)
    return tuple(d[i].reshape(n, h, w, 1) for i in range(k))
